# baseline (speedup 1.0000x reference)
import jax
import jax.numpy as jnp
from jax.experimental import pallas as pl
from jax.experimental.pallas import tpu as pltpu

A = -1.0
H = 0.0
K = 1.0
S = 64.0

_HALF_PI = 1.5707963267948966
_PI = 3.141592653589793


def _asin_poly(z):
    p = 4.2163199048e-2
    p = p * z + 2.4181311049e-2
    p = p * z + 4.5470025998e-2
    p = p * z + 7.4953002686e-2
    p = p * z + 1.6666752422e-1
    return p


def _acos(x):
    ax = jnp.abs(x)
    z_s = x * x
    acos_small = _HALF_PI - (x + x * z_s * _asin_poly(z_s))
    t = 0.5 * (1.0 - ax)
    s = jnp.sqrt(t)
    r = 2.0 * (s + s * t * _asin_poly(t))
    acos_big = jnp.where(x > 0.0, r, _PI - r)
    return jnp.where(ax > 0.5, acos_big, acos_small)


def _tc_fused(logits, labels, block_rows=16):
    B, V = logits.shape
    lab2 = labels.reshape(B, 1)

    def body(x_ref, lab_ref, o_ref):
        i = pl.program_id(0)
        lab = lab_ref[pl.ds(i * block_rows, block_rows), :]
        x = x_ref[...]
        col = jax.lax.broadcasted_iota(jnp.int32, x.shape, 1)
        mask = col == lab
        t = jnp.sum(jnp.where(mask, x, 0.0), axis=1, keepdims=True)
        theta = _acos(t)
        newv = (A * (theta - H) ** 2 + K) * S
        o_ref[...] = jnp.where(mask, newv, x * S)

    return pl.pallas_call(
        body,
        grid=(B // block_rows,),
        in_specs=[
            pl.BlockSpec((block_rows, V), lambda i: (i, 0)),
            pl.BlockSpec((B, 1), lambda i: (0, 0)),
        ],
        out_specs=pl.BlockSpec((block_rows, V), lambda i: (i, 0)),
        out_shape=jax.ShapeDtypeStruct((B, V), jnp.float32),
        compiler_params=pltpu.CompilerParams(
            dimension_semantics=("parallel",)),
    )(logits, lab2)


def kernel(logits, labels):
    return _tc_fused(logits, labels)
